# baseline (speedup 1.0000x reference)
import jax
import jax.numpy as jnp
import numpy as np
from jax.experimental import pallas as pl
from jax.experimental.pallas import tpu as pltpu

NP = 55
NJ = 56
NQ = NJ // 4
NQP = 16
RP = NP * NQ
NF = 32
NH = 32
CR = 5.0
BB = 4
RT = BB * RP
BF = jnp.bfloat16
F32 = jnp.float32


def _consts():
    R16 = np.zeros((RT, BB * NJ), np.float32)
    R55 = np.zeros((RT, BB * NQP), np.float32)
    S2 = np.zeros((BB * NP, RT), np.float32)
    maskp = np.zeros((RT, 4 * NF), np.float32)
    wmask4 = np.zeros((RT, 4), np.float32)
    selmask = np.zeros((RT, 4), np.float32)
    for bb in range(BB):
        for i in range(NP):
            for q in range(NQ):
                r = bb * RP + i * NQ + q
                R16[r, bb * NJ + i] = 1.0
                R55[r, bb * NQP + q] = 1.0
                S2[bb * NP + i, r] = 1.0
                for g in range(4):
                    j = 4 * q + g
                    valid = (j != i) and (j < NP)
                    if valid:
                        maskp[r, NF * g:NF * (g + 1)] = 1.0
                        wmask4[r, g] = 1.0
                    if j < i:
                        selmask[r, g] = 1.0
    F16 = np.zeros((16, 8), np.float32)
    for d in range(3):
        for g in range(4):
            F16[4 * d + g, d] = 1.0
    return R16, R55, S2, maskp, wmask4, selmask, F16


_R16, _R55, _S2, _MASKP, _WMASK4, _SELMASK, _F16 = _consts()


def _body(xi16_ref, xj16_ref, xn55_ref, h55_ref, hp_ref, hB_ref,
          dsbP_ref, dsSP_ref,
          R16_ref, R55_ref, S2_ref, maskp_ref,
          wmask4_ref, selmask_ref, F16_ref,
          Wh1t_ref, Wh2b_ref, W12_ref,
          We2b_ref, be2t_ref, Wab_ref, ba_ref, G4_ref,
          Wc1b_ref, bc1t_ref, Wc2b_ref, F4_ref,
          Wn1_ref, bn1_ref, Wn2_ref, bn2_ref,
          xo_ref, ho_ref):
    fdot = lambda a, b: jnp.dot(a, b, preferred_element_type=F32)
    LOG2E = 1.4426950408889634

    def _sig(v):
        one = jnp.asarray(1.0, v.dtype)
        l2e = jnp.asarray(-LOG2E, v.dtype)
        return one / (one + jnp.exp2(v * l2e))

    def _silu(v):
        return v * _sig(v)

    dvec = xi16_ref[0] - xj16_ref[0]
    sq = dvec * dvec
    d2 = sq[:, 0:4] + sq[:, 4:8] + sq[:, 8:12] + jnp.asarray(1e-6, BF)
    d = jnp.sqrt(d2)

    sel = selmask_ref[...]
    dsf = dsSP_ref[0] + sel * (dsbP_ref[0] - dsSP_ref[0])
    ds2 = dsf * dsf
    gf = jnp.concatenate([d2, ds2, jnp.ones((RT, 4), BF)], axis=1)

    Ptile = fdot(hp_ref[0], Wh1t_ref[...])
    Ppack = fdot(R16_ref[...], Ptile.astype(BF))
    QB = fdot(hB_ref[0], Wh2b_ref[...])
    Qpack = fdot(R55_ref[...], QB.astype(BF))
    z = Ppack + Qpack + fdot(gf, W12_ref[...])
    m1 = _silu(z.astype(BF))

    a = fdot(m1, We2b_ref[...]) + be2t_ref[...]
    m2 = _silu(a.astype(BF))
    att = _sig((fdot(m2, Wab_ref[...]) + ba_ref[...]).astype(BF))
    attb = fdot(att, G4_ref[...])
    m3 = m2 * attb.astype(BF)

    cpre = fdot(m3, Wc1b_ref[...]) + bc1t_ref[...]
    c = _silu(cpre.astype(BF))
    cw = jnp.tanh(fdot(c, Wc2b_ref[...]).astype(BF))
    one = jnp.asarray(1.0, BF)
    wb = cw / (d + one) * wmask4_ref[...]
    w16 = jnp.concatenate([wb, wb, wb, wb], axis=1)
    prod = dvec * w16

    m3m = m3 * maskp_ref[...]
    agg_in = jnp.concatenate([m3m, prod], axis=1)
    AGG = fdot(S2_ref[...], agg_in)
    mi128 = AGG[:, :4 * NF]
    U = AGG[:, 4 * NF:]
    upd = fdot(U, F16_ref[...])
    xo_ref[0] = xn55_ref[0] + CR * upd

    m_i = fdot(mi128.astype(BF), F4_ref[...])
    hm = jnp.concatenate([h55_ref[0], m_i], axis=1)
    t = fdot(hm.astype(BF), Wn1_ref[...]) + bn1_ref[...]
    t = _silu(t)
    hu = fdot(t.astype(BF), Wn2_ref[...]) + bn2_ref[...]
    ho_ref[0] = h55_ref[0] + hu


def kernel(x, h, d_static, We1, be1, We2, be2, Wn1, bn1, Wn2, bn2,
           Wc1, bc1, Wc2, Wa, ba):
    B = x.shape[0]
    G = B // BB
    xv = x.reshape(B, NP, 3)
    xvb = xv.astype(BF)
    xvpb = jnp.pad(xvb, ((0, 0), (0, NJ - NP), (0, 0)))

    xi16s = jnp.pad(jnp.repeat(xvb, 4, axis=2), ((0, 0), (0, 0), (0, 4)))
    xi16 = (jnp.broadcast_to(xi16s[:, :, None, :], (B, NP, NQ, 16))
            .reshape(G, RT, 16))
    xj16s = jnp.pad(
        xvpb.reshape(B, NQ, 4, 3).transpose(0, 1, 3, 2).reshape(B, NQ, 12),
        ((0, 0), (0, 0), (0, 4)))
    xj16 = (jnp.broadcast_to(xj16s[:, None, :, :], (B, NP, NQ, 16))
            .reshape(G, RT, 16))

    xn55 = jnp.pad(xv, ((0, 0), (0, 0), (0, 5))).reshape(G, BB * NP, 8)
    h55 = h.reshape(G, BB * NP, NF)
    hpb = jnp.pad(h.astype(BF), ((0, 0), (0, NJ - NP), (0, 0)))
    hp = hpb.reshape(G, BB * NJ, NF)
    hB = jnp.pad(hpb.reshape(B, NQ, 4 * NF),
                 ((0, 0), (0, NQP - NQ), (0, 0))).reshape(G, BB * NQP, 4 * NF)
    dspb = jnp.pad(d_static, ((0, 0), (0, 0), (0, NJ - (NP - 1)))).astype(BF)
    dssb = jnp.concatenate(
        [jnp.zeros((B, NP, 1), BF), dspb[:, :, :NJ - 1]], axis=2)
    dsbP = dspb.reshape(G, RT, 4)
    dsSP = dssb.reshape(G, RT, 4)

    eye4 = jnp.eye(4, dtype=F32)
    Wh1 = We1[:NF]
    Wh2 = We1[NF:2 * NF]
    wd = We1[2 * NF:2 * NF + 1]
    ws = We1[2 * NF + 1:2 * NF + 2]
    W12 = jnp.concatenate(
        [jnp.kron(eye4, wd), jnp.kron(eye4, ws),
         jnp.tile(be1.reshape(1, NH), (1, 4)),
         jnp.zeros((3, 4 * NH), F32)], axis=0)
    consts = dict(
        R16=jnp.asarray(_R16, BF), R55=jnp.asarray(_R55, BF),
        S2=jnp.asarray(_S2, BF), maskp=jnp.asarray(_MASKP, BF),
        wmask4=jnp.asarray(_WMASK4, BF), selmask=jnp.asarray(_SELMASK, BF),
        F16=jnp.asarray(_F16),
        Wh1t=jnp.tile(Wh1, (1, 4)).astype(BF),
        Wh2b=jnp.kron(eye4, Wh2).astype(BF),
        W12=W12.astype(BF),
        We2b=jnp.kron(eye4, We2).astype(BF),
        be2t=jnp.tile(be2.reshape(1, NF), (1, 4)),
        Wab=jnp.kron(eye4, Wa).astype(BF),
        ba=ba.reshape(1, 1),
        G4=jnp.kron(eye4, jnp.ones((1, NF), F32)).astype(BF),
        Wc1b=jnp.kron(eye4, Wc1).astype(BF),
        bc1t=jnp.tile(bc1.reshape(1, NH), (1, 4)),
        Wc2b=jnp.kron(eye4, Wc2).astype(BF),
        F4=jnp.tile(jnp.eye(NF, dtype=F32), (4, 1)).astype(BF),
        Wn1=Wn1.astype(BF), bn1=bn1.reshape(1, NH),
        Wn2=Wn2.astype(BF), bn2=bn2.reshape(1, NF),
    )

    def batch_spec(shp):
        return pl.BlockSpec((1,) + shp, lambda b: (b, 0, 0))

    def const_spec(arr):
        return pl.BlockSpec(arr.shape, lambda b: (0, 0))

    batch_args = (xi16, xj16, xn55, h55, hp, hB, dsbP, dsSP)
    batch_shapes = ((RT, 16), (RT, 16), (BB * NP, 8), (BB * NP, NF),
                    (BB * NJ, NF), (BB * NQP, 4 * NF), (RT, 4), (RT, 4))
    const_args = tuple(consts.values())

    in_specs = ([batch_spec(s) for s in batch_shapes]
                + [const_spec(a) for a in const_args])
    out_specs = (batch_spec((BB * NP, 8)), batch_spec((BB * NP, NF)))
    out_shape = (jax.ShapeDtypeStruct((G, BB * NP, 8), F32),
                 jax.ShapeDtypeStruct((G, BB * NP, NF), F32))

    xo, ho = pl.pallas_call(
        _body, grid=(G,), in_specs=in_specs, out_specs=out_specs,
        out_shape=out_shape,
        compiler_params=pltpu.CompilerParams(
            dimension_semantics=("parallel",)))(*batch_args, *const_args)
    return (xo.reshape(B, NP, 8)[:, :, :3], ho.reshape(B, NP, NF))


# speedup vs baseline: 1.6076x; 1.6076x over previous
import jax
import jax.numpy as jnp
import numpy as np
from jax.experimental import pallas as pl
from jax.experimental.pallas import tpu as pltpu

NP = 55
NJ = 56
NQ = NJ // 4
NQP = 16
RP = NP * NQ
NF = 32
NH = 32
CR = 5.0
BB = 4
RT = BB * RP
BF = jnp.bfloat16
F32 = jnp.float32


def _consts():
    R16 = np.zeros((RT, BB * NJ), np.float32)
    R55 = np.zeros((RT, BB * NQP), np.float32)
    S2 = np.zeros((BB * NP, RT), np.float32)
    maskp = np.zeros((RT, 4 * NF), np.float32)
    wmask4 = np.zeros((RT, 4), np.float32)
    selmask = np.zeros((RT, 4), np.float32)
    for bb in range(BB):
        for i in range(NP):
            for q in range(NQ):
                r = bb * RP + i * NQ + q
                R16[r, bb * NJ + i] = 1.0
                R55[r, bb * NQP + q] = 1.0
                S2[bb * NP + i, r] = 1.0
                for g in range(4):
                    j = 4 * q + g
                    valid = (j != i) and (j < NP)
                    if valid:
                        maskp[r, NF * g:NF * (g + 1)] = 1.0
                        wmask4[r, g] = 1.0
                    if j < i:
                        selmask[r, g] = 1.0
    F16 = np.zeros((16, 8), np.float32)
    for d in range(3):
        for g in range(4):
            F16[4 * d + g, d] = 1.0
    return R16, R55, S2, maskp, wmask4, selmask, F16


_R16, _R55, _S2, _MASKP, _WMASK4, _SELMASK, _F16 = _consts()


def _body(xn16_ref, xjB16_ref, xn55_ref, h55_ref, hp_ref, hB_ref,
          dsbP_ref, dsSP_ref,
          R16_ref, R55_ref, S2_ref, maskp_ref,
          wmask4_ref, selmask_ref, F16_ref,
          Wh1t_ref, Wh2b_ref, W12_ref,
          We2b_ref, be2t_ref, Wab_ref, ba_ref, G4_ref,
          Wc1b_ref, bc1t_ref, Wc2b_ref, F4_ref,
          Wn1_ref, bn1_ref, Wn2_ref, bn2_ref,
          xo_ref, ho_ref):
    fdot = lambda a, b: jnp.dot(a, b, preferred_element_type=F32)
    LOG2E = 1.4426950408889634

    def _sig(v):
        one = jnp.asarray(1.0, v.dtype)
        l2e = jnp.asarray(-LOG2E, v.dtype)
        return one / (one + jnp.exp2(v * l2e))

    def _silu(v):
        return v * _sig(v)

    Ptile = fdot(hp_ref[0], Wh1t_ref[...])
    Pcat = jnp.concatenate([Ptile.astype(BF), xn16_ref[0]], axis=1)
    P2 = fdot(R16_ref[...], Pcat)
    QB = fdot(hB_ref[0], Wh2b_ref[...])
    Qcat = jnp.concatenate([QB.astype(BF), xjB16_ref[0]], axis=1)
    Q2 = fdot(R55_ref[...], Qcat)

    dvec = (P2[:, 128:] - Q2[:, 128:]).astype(BF)
    sq = dvec * dvec
    d2 = sq[:, 0:4] + sq[:, 4:8] + sq[:, 8:12] + jnp.asarray(1e-6, BF)
    d = jnp.sqrt(d2)

    sel = selmask_ref[...]
    dsf = dsSP_ref[0] + sel * (dsbP_ref[0] - dsSP_ref[0])
    ds2 = dsf * dsf
    gf = jnp.concatenate([d2, ds2, jnp.ones((RT, 4), BF)], axis=1)

    z = P2[:, :128] + Q2[:, :128] + fdot(gf, W12_ref[...])
    m1 = _silu(z.astype(BF))

    a = fdot(m1, We2b_ref[...]) + be2t_ref[...]
    m2 = _silu(a.astype(BF))
    att = _sig((fdot(m2, Wab_ref[...]) + ba_ref[...]).astype(BF))
    attb = fdot(att, G4_ref[...])
    m3 = m2 * attb.astype(BF)

    cpre = fdot(m3, Wc1b_ref[...]) + bc1t_ref[...]
    c = _silu(cpre.astype(BF))
    cw = jnp.tanh(fdot(c, Wc2b_ref[...]).astype(BF))
    one = jnp.asarray(1.0, BF)
    wb = cw / (d + one) * wmask4_ref[...]
    w16 = jnp.concatenate([wb, wb, wb, wb], axis=1)
    prod = dvec * w16

    m3m = m3 * maskp_ref[...]
    agg_in = jnp.concatenate([m3m, prod], axis=1)
    AGG = fdot(S2_ref[...], agg_in)
    mi128 = AGG[:, :4 * NF]
    U = AGG[:, 4 * NF:]
    upd = fdot(U, F16_ref[...])
    xo_ref[0] = xn55_ref[0] + CR * upd

    m_i = fdot(mi128.astype(BF), F4_ref[...])
    hm = jnp.concatenate([h55_ref[0], m_i], axis=1)
    t = fdot(hm.astype(BF), Wn1_ref[...]) + bn1_ref[...]
    t = _silu(t)
    hu = fdot(t.astype(BF), Wn2_ref[...]) + bn2_ref[...]
    ho_ref[0] = h55_ref[0] + hu


def kernel(x, h, d_static, We1, be1, We2, be2, Wn1, bn1, Wn2, bn2,
           Wc1, bc1, Wc2, Wa, ba):
    B = x.shape[0]
    G = B // BB
    xv = x.reshape(B, NP, 3)
    xvp = jnp.pad(xv, ((0, 0), (0, NJ - NP), (0, 0)))

    xn16 = jnp.pad(jnp.repeat(xvp, 4, axis=2),
                   ((0, 0), (0, 0), (0, 4))).reshape(G, BB * NJ, 16)
    xjB16 = jnp.pad(
        xvp.reshape(B, NQ, 4, 3).transpose(0, 1, 3, 2).reshape(B, NQ, 12),
        ((0, 0), (0, NQP - NQ), (0, 4))).reshape(G, BB * NQP, 16)

    xn55 = jnp.pad(xv, ((0, 0), (0, 0), (0, 5))).reshape(G, BB * NP, 8)
    h55 = h.reshape(G, BB * NP, NF)
    hp = jnp.pad(h, ((0, 0), (0, NJ - NP), (0, 0))).reshape(G, BB * NJ, NF)
    hB = jnp.pad(
        jnp.pad(h, ((0, 0), (0, NJ - NP), (0, 0)))
        .reshape(B, NQ, 4 * NF), ((0, 0), (0, NQP - NQ), (0, 0))
    ).reshape(G, BB * NQP, 4 * NF)
    dsp = jnp.pad(d_static, ((0, 0), (0, 0), (0, NJ - (NP - 1))))
    dss = jnp.concatenate(
        [jnp.zeros((B, NP, 1), F32), dsp[:, :, :NJ - 1]], axis=2)
    dsbP = dsp.reshape(G, RT, 4).astype(BF)
    dsSP = dss.reshape(G, RT, 4).astype(BF)

    eye4 = jnp.eye(4, dtype=F32)
    Wh1 = We1[:NF]
    Wh2 = We1[NF:2 * NF]
    wd = We1[2 * NF:2 * NF + 1]
    ws = We1[2 * NF + 1:2 * NF + 2]
    W12 = jnp.concatenate(
        [jnp.kron(eye4, wd), jnp.kron(eye4, ws),
         jnp.tile(be1.reshape(1, NH), (1, 4)),
         jnp.zeros((3, 4 * NH), F32)], axis=0)
    consts = dict(
        R16=jnp.asarray(_R16, BF), R55=jnp.asarray(_R55, BF),
        S2=jnp.asarray(_S2, BF), maskp=jnp.asarray(_MASKP, BF),
        wmask4=jnp.asarray(_WMASK4, BF), selmask=jnp.asarray(_SELMASK, BF),
        F16=jnp.asarray(_F16),
        Wh1t=jnp.tile(Wh1, (1, 4)).astype(BF),
        Wh2b=jnp.kron(eye4, Wh2).astype(BF),
        W12=W12.astype(BF),
        We2b=jnp.kron(eye4, We2).astype(BF),
        be2t=jnp.tile(be2.reshape(1, NF), (1, 4)),
        Wab=jnp.kron(eye4, Wa).astype(BF),
        ba=ba.reshape(1, 1),
        G4=jnp.kron(eye4, jnp.ones((1, NF), F32)).astype(BF),
        Wc1b=jnp.kron(eye4, Wc1).astype(BF),
        bc1t=jnp.tile(bc1.reshape(1, NH), (1, 4)),
        Wc2b=jnp.kron(eye4, Wc2).astype(BF),
        F4=jnp.tile(jnp.eye(NF, dtype=F32), (4, 1)).astype(BF),
        Wn1=Wn1.astype(BF), bn1=bn1.reshape(1, NH),
        Wn2=Wn2.astype(BF), bn2=bn2.reshape(1, NF),
    )

    def batch_spec(shp):
        return pl.BlockSpec((1,) + shp, lambda b: (b, 0, 0))

    def const_spec(arr):
        return pl.BlockSpec(arr.shape, lambda b: (0, 0))

    batch_args = (xn16.astype(BF), xjB16.astype(BF), xn55, h55,
                  hp.astype(BF), hB.astype(BF), dsbP, dsSP)
    batch_shapes = ((BB * NJ, 16), (BB * NQP, 16), (BB * NP, 8), (BB * NP, NF),
                    (BB * NJ, NF), (BB * NQP, 4 * NF), (RT, 4), (RT, 4))
    const_args = tuple(consts.values())

    in_specs = ([batch_spec(s) for s in batch_shapes]
                + [const_spec(a) for a in const_args])
    out_specs = (batch_spec((BB * NP, 8)), batch_spec((BB * NP, NF)))
    out_shape = (jax.ShapeDtypeStruct((G, BB * NP, 8), F32),
                 jax.ShapeDtypeStruct((G, BB * NP, NF), F32))

    xo, ho = pl.pallas_call(
        _body, grid=(G,), in_specs=in_specs, out_specs=out_specs,
        out_shape=out_shape,
        compiler_params=pltpu.CompilerParams(
            dimension_semantics=("parallel",)))(*batch_args, *const_args)
    return (xo.reshape(B, NP, 8)[:, :, :3], ho.reshape(B, NP, NF))


# speedup vs baseline: 1.7710x; 1.1017x over previous
import jax
import jax.numpy as jnp
import numpy as np
from jax.experimental import pallas as pl
from jax.experimental.pallas import tpu as pltpu

NP = 55
NJ = 56
NQ = NJ // 4
NQP = 16
RP = NP * NQ
NF = 32
NH = 32
CR = 5.0
BB = 4
RT = BB * RP
BF = jnp.bfloat16
F32 = jnp.float32


def _consts():
    R16 = np.zeros((RT, BB * NJ), np.float32)
    R55 = np.zeros((RT, BB * NQP), np.float32)
    S2 = np.zeros((BB * NP, RT), np.float32)
    maskp = np.zeros((RT, 4 * NF), np.float32)
    wmask4 = np.zeros((RT, 4), np.float32)
    selmask = np.zeros((RT, 4), np.float32)
    for bb in range(BB):
        for i in range(NP):
            for q in range(NQ):
                r = bb * RP + i * NQ + q
                R16[r, bb * NJ + i] = 1.0
                R55[r, bb * NQP + q] = 1.0
                S2[bb * NP + i, r] = 1.0
                for g in range(4):
                    j = 4 * q + g
                    valid = (j != i) and (j < NP)
                    if valid:
                        maskp[r, NF * g:NF * (g + 1)] = 1.0
                        wmask4[r, g] = 1.0
                    if j < i:
                        selmask[r, g] = 1.0
    F16 = np.zeros((16, 8), np.float32)
    for d in range(3):
        for g in range(4):
            F16[4 * d + g, d] = 1.0
    return R16, R55, S2, maskp, wmask4, selmask, F16


_R16, _R55, _S2, _MASKP, _WMASK4, _SELMASK, _F16 = _consts()


def _body(xn16_ref, xjB16_ref, xn55_ref, h55_ref, hp_ref, hB_ref,
          dsbP_ref,
          R16_ref, R55_ref, S2_ref, maskp_ref,
          wmask4_ref, selmask_ref, F16_ref,
          Wh1t_ref, Wh2b_ref, W12_ref,
          We2b_ref, be2t_ref, Wab_ref, ba_ref, G4_ref,
          Wc1b_ref, bc1t_ref, Wc2b_ref, F4_ref,
          Wn1_ref, bn1_ref, Wn2_ref, bn2_ref,
          xo_ref, ho_ref):
    fdot = lambda a, b: jnp.dot(a, b, preferred_element_type=F32)
    LOG2E = 1.4426950408889634

    def _sig(v):
        one = jnp.asarray(1.0, v.dtype)
        l2e = jnp.asarray(-LOG2E, v.dtype)
        return one / (one + jnp.exp2(v * l2e))

    def _silu(v):
        return v * _sig(v)

    Ptile = fdot(hp_ref[0], Wh1t_ref[...])
    Pcat = jnp.concatenate([Ptile.astype(BF), xn16_ref[0]], axis=1)
    P2 = fdot(R16_ref[...], Pcat)
    QB = fdot(hB_ref[0], Wh2b_ref[...])
    Qcat = jnp.concatenate([QB.astype(BF), xjB16_ref[0]], axis=1)
    Q2 = fdot(R55_ref[...], Qcat)

    dvec = (P2[:, 128:] - Q2[:, 128:]).astype(BF)
    sq = dvec * dvec
    d2 = sq[:, 0:4] + sq[:, 4:8] + sq[:, 8:12] + jnp.asarray(1e-6, BF)
    d = jnp.sqrt(d2)

    dsb = dsbP_ref[0]
    prev3 = jnp.concatenate(
        [jnp.zeros((1, 1), BF), dsb[:-1, 3:4]], axis=0)
    dsS = jnp.concatenate([prev3, dsb[:, 0:3]], axis=1)
    sel = selmask_ref[...]
    dsf = dsS + sel * (dsb - dsS)
    ds2 = dsf * dsf
    gf = jnp.concatenate([d2, ds2, jnp.ones((RT, 4), BF)], axis=1)

    z = P2[:, :128] + Q2[:, :128] + fdot(gf, W12_ref[...])
    m1 = _silu(z.astype(BF))

    a = fdot(m1, We2b_ref[...]) + be2t_ref[...]
    m2 = _silu(a.astype(BF))
    att = _sig((fdot(m2, Wab_ref[...]) + ba_ref[...]).astype(BF))
    attb = fdot(att, G4_ref[...])
    m3 = m2 * attb.astype(BF)

    cpre = fdot(m3, Wc1b_ref[...]) + bc1t_ref[...]
    c = _silu(cpre.astype(BF))
    cw = jnp.tanh(fdot(c, Wc2b_ref[...]).astype(BF))
    one = jnp.asarray(1.0, BF)
    wb = cw / (d + one) * wmask4_ref[...]
    w16 = jnp.concatenate([wb, wb, wb, wb], axis=1)
    prod = dvec * w16

    m3m = m3 * maskp_ref[...]
    agg_in = jnp.concatenate([m3m, prod], axis=1)
    AGG = fdot(S2_ref[...], agg_in)
    mi128 = AGG[:, :4 * NF]
    U = AGG[:, 4 * NF:]
    upd = fdot(U, F16_ref[...])
    xo_ref[0] = xn55_ref[0] + CR * upd

    m_i = fdot(mi128.astype(BF), F4_ref[...])
    hm = jnp.concatenate([h55_ref[0], m_i], axis=1)
    t = fdot(hm.astype(BF), Wn1_ref[...]) + bn1_ref[...]
    t = _silu(t)
    hu = fdot(t.astype(BF), Wn2_ref[...]) + bn2_ref[...]
    ho_ref[0] = h55_ref[0] + hu


def kernel(x, h, d_static, We1, be1, We2, be2, Wn1, bn1, Wn2, bn2,
           Wc1, bc1, Wc2, Wa, ba):
    B = x.shape[0]
    G = B // BB
    xv = x.reshape(B, NP, 3)
    xvp = jnp.pad(xv, ((0, 0), (0, NJ - NP), (0, 0)))

    xn16 = jnp.pad(jnp.repeat(xvp, 4, axis=2),
                   ((0, 0), (0, 0), (0, 4))).reshape(G, BB * NJ, 16)
    xjB16 = jnp.pad(
        xvp.reshape(B, NQ, 4, 3).transpose(0, 1, 3, 2).reshape(B, NQ, 12),
        ((0, 0), (0, NQP - NQ), (0, 4))).reshape(G, BB * NQP, 16)

    xn55 = jnp.pad(xv, ((0, 0), (0, 0), (0, 5))).reshape(G, BB * NP, 8)
    h55 = h.reshape(G, BB * NP, NF)
    hp = jnp.pad(h, ((0, 0), (0, NJ - NP), (0, 0))).reshape(G, BB * NJ, NF)
    hB = jnp.pad(
        jnp.pad(h, ((0, 0), (0, NJ - NP), (0, 0)))
        .reshape(B, NQ, 4 * NF), ((0, 0), (0, NQP - NQ), (0, 0))
    ).reshape(G, BB * NQP, 4 * NF)
    dsp = jnp.pad(d_static, ((0, 0), (0, 0), (0, NJ - (NP - 1))))
    dsbP = dsp.reshape(G, RT, 4).astype(BF)

    eye4 = jnp.eye(4, dtype=F32)
    Wh1 = We1[:NF]
    Wh2 = We1[NF:2 * NF]
    wd = We1[2 * NF:2 * NF + 1]
    ws = We1[2 * NF + 1:2 * NF + 2]
    W12 = jnp.concatenate(
        [jnp.kron(eye4, wd), jnp.kron(eye4, ws),
         jnp.tile(be1.reshape(1, NH), (1, 4)),
         jnp.zeros((3, 4 * NH), F32)], axis=0)
    consts = dict(
        R16=jnp.asarray(_R16, BF), R55=jnp.asarray(_R55, BF),
        S2=jnp.asarray(_S2, BF), maskp=jnp.asarray(_MASKP, BF),
        wmask4=jnp.asarray(_WMASK4, BF), selmask=jnp.asarray(_SELMASK, BF),
        F16=jnp.asarray(_F16),
        Wh1t=jnp.tile(Wh1, (1, 4)).astype(BF),
        Wh2b=jnp.kron(eye4, Wh2).astype(BF),
        W12=W12.astype(BF),
        We2b=jnp.kron(eye4, We2).astype(BF),
        be2t=jnp.tile(be2.reshape(1, NF), (1, 4)),
        Wab=jnp.kron(eye4, Wa).astype(BF),
        ba=ba.reshape(1, 1),
        G4=jnp.kron(eye4, jnp.ones((1, NF), F32)).astype(BF),
        Wc1b=jnp.kron(eye4, Wc1).astype(BF),
        bc1t=jnp.tile(bc1.reshape(1, NH), (1, 4)),
        Wc2b=jnp.kron(eye4, Wc2).astype(BF),
        F4=jnp.tile(jnp.eye(NF, dtype=F32), (4, 1)).astype(BF),
        Wn1=Wn1.astype(BF), bn1=bn1.reshape(1, NH),
        Wn2=Wn2.astype(BF), bn2=bn2.reshape(1, NF),
    )

    def batch_spec(shp):
        return pl.BlockSpec((1,) + shp, lambda b: (b, 0, 0))

    def const_spec(arr):
        return pl.BlockSpec(arr.shape, lambda b: (0, 0))

    batch_args = (xn16.astype(BF), xjB16.astype(BF), xn55, h55,
                  hp.astype(BF), hB.astype(BF), dsbP)
    batch_shapes = ((BB * NJ, 16), (BB * NQP, 16), (BB * NP, 8), (BB * NP, NF),
                    (BB * NJ, NF), (BB * NQP, 4 * NF), (RT, 4))
    const_args = tuple(consts.values())

    in_specs = ([batch_spec(s) for s in batch_shapes]
                + [const_spec(a) for a in const_args])
    out_specs = (batch_spec((BB * NP, 8)), batch_spec((BB * NP, NF)))
    out_shape = (jax.ShapeDtypeStruct((G, BB * NP, 8), F32),
                 jax.ShapeDtypeStruct((G, BB * NP, NF), F32))

    xo, ho = pl.pallas_call(
        _body, grid=(G,), in_specs=in_specs, out_specs=out_specs,
        out_shape=out_shape,
        compiler_params=pltpu.CompilerParams(
            dimension_semantics=("parallel",)))(*batch_args, *const_args)
    return (xo.reshape(B, NP, 8)[:, :, :3], ho.reshape(B, NP, NF))


# speedup vs baseline: 1.8187x; 1.0269x over previous
import jax
import jax.numpy as jnp
import numpy as np
from jax.experimental import pallas as pl
from jax.experimental.pallas import tpu as pltpu

NP = 55
NJ = 56
NQ = NJ // 4
NQP = 16
RP = NP * NQ
NF = 32
NH = 32
CR = 5.0
BB = 4
RT = BB * RP
BF = jnp.bfloat16
F32 = jnp.float32


def _consts():
    R16 = np.zeros((RT, BB * NJ), np.float32)
    R55 = np.zeros((RT, BB * NQP), np.float32)
    S2 = np.zeros((BB * NP, RT), np.float32)
    maskp = np.zeros((RT, 4 * NF), np.float32)
    wmask4 = np.zeros((RT, 4), np.float32)
    selmask = np.zeros((RT, 4), np.float32)
    for bb in range(BB):
        for i in range(NP):
            for q in range(NQ):
                r = bb * RP + i * NQ + q
                R16[r, bb * NJ + i] = 1.0
                R55[r, bb * NQP + q] = 1.0
                S2[bb * NP + i, r] = 1.0
                for g in range(4):
                    j = 4 * q + g
                    valid = (j != i) and (j < NP)
                    if valid:
                        maskp[r, NF * g:NF * (g + 1)] = 1.0
                        wmask4[r, g] = 1.0
                    if j < i:
                        selmask[r, g] = 1.0
    F16 = np.zeros((16, 8), np.float32)
    for d in range(3):
        for g in range(4):
            F16[4 * d + g, d] = 1.0
    return R16, R55, S2, maskp, wmask4, selmask, F16


_R16, _R55, _S2, _MASKP, _WMASK4, _SELMASK, _F16 = _consts()


def _body(xn16_ref, xjB16_ref, xn55_ref, h55_ref, hp_ref, hB_ref,
          dsbP_ref,
          R16_ref, R55_ref, S2_ref, maskp_ref,
          wmask4_ref, selmask_ref, F16_ref,
          Wh1t_ref, Wh2b_ref, W12_ref,
          We2b_ref, be2t_ref, Wab_ref, ba_ref, G4_ref,
          Wc1b_ref, bc1t_ref, Wc2b_ref, F4_ref,
          Wn1_ref, bn1_ref, Wn2_ref, bn2_ref,
          xo_ref, ho_ref):
    fdot = lambda a, b: jnp.dot(a, b, preferred_element_type=F32)
    LOG2E = 1.4426950408889634

    def _sig(v):
        half = jnp.asarray(0.5, v.dtype)
        one = jnp.asarray(1.0, v.dtype)
        return half * (one + jnp.tanh(v * half))

    def _silu(v):
        half = jnp.asarray(0.5, v.dtype)
        one = jnp.asarray(1.0, v.dtype)
        y = v * half
        return y * (one + jnp.tanh(y))

    Ptile = fdot(hp_ref[0], Wh1t_ref[...])
    Pcat = jnp.concatenate([Ptile.astype(BF), xn16_ref[0]], axis=1)
    P2 = fdot(R16_ref[...], Pcat)
    QB = fdot(hB_ref[0], Wh2b_ref[...])
    Qcat = jnp.concatenate([QB.astype(BF), xjB16_ref[0]], axis=1)
    Q2 = fdot(R55_ref[...], Qcat)

    dvec = (P2[:, 128:] - Q2[:, 128:]).astype(BF)
    sq = dvec * dvec
    d2 = sq[:, 0:4] + sq[:, 4:8] + sq[:, 8:12] + jnp.asarray(1e-6, BF)
    d = jnp.sqrt(d2)

    dsb = dsbP_ref[0]
    prev3 = jnp.concatenate(
        [jnp.zeros((1, 1), BF), dsb[:-1, 3:4]], axis=0)
    dsS = jnp.concatenate([prev3, dsb[:, 0:3]], axis=1)
    sel = selmask_ref[...]
    dsf = dsS + sel * (dsb - dsS)
    ds2 = dsf * dsf
    gf = jnp.concatenate([d2, ds2, jnp.ones((RT, 4), BF)], axis=1)

    z = P2[:, :128] + Q2[:, :128] + fdot(gf, W12_ref[...])
    m1 = _silu(z.astype(BF))

    a = fdot(m1, We2b_ref[...]) + be2t_ref[...]
    m2 = _silu(a.astype(BF))
    att = _sig((fdot(m2, Wab_ref[...]) + ba_ref[...]).astype(BF))
    attb = fdot(att, G4_ref[...])
    m3 = m2 * attb.astype(BF)

    cpre = fdot(m3, Wc1b_ref[...]) + bc1t_ref[...]
    c = _silu(cpre.astype(BF))
    cw = jnp.tanh(fdot(c, Wc2b_ref[...]).astype(BF))
    one = jnp.asarray(1.0, BF)
    wb = cw / (d + one) * wmask4_ref[...]
    w16 = jnp.concatenate([wb, wb, wb, wb], axis=1)
    prod = dvec * w16

    m3m = m3 * maskp_ref[...]
    agg_in = jnp.concatenate([m3m, prod], axis=1)
    AGG = fdot(S2_ref[...], agg_in)
    mi128 = AGG[:, :4 * NF]
    U = AGG[:, 4 * NF:]
    upd = fdot(U, F16_ref[...])
    xo_ref[0] = xn55_ref[0] + CR * upd

    m_i = fdot(mi128.astype(BF), F4_ref[...])
    hm = jnp.concatenate([h55_ref[0], m_i], axis=1)
    t = fdot(hm.astype(BF), Wn1_ref[...]) + bn1_ref[...]
    t = _silu(t)
    hu = fdot(t.astype(BF), Wn2_ref[...]) + bn2_ref[...]
    ho_ref[0] = h55_ref[0] + hu


def kernel(x, h, d_static, We1, be1, We2, be2, Wn1, bn1, Wn2, bn2,
           Wc1, bc1, Wc2, Wa, ba):
    B = x.shape[0]
    G = B // BB
    xv = x.reshape(B, NP, 3)
    xvp = jnp.pad(xv, ((0, 0), (0, NJ - NP), (0, 0)))

    xn16 = jnp.pad(jnp.repeat(xvp, 4, axis=2),
                   ((0, 0), (0, 0), (0, 4))).reshape(G, BB * NJ, 16)
    xjB16 = jnp.pad(
        xvp.reshape(B, NQ, 4, 3).transpose(0, 1, 3, 2).reshape(B, NQ, 12),
        ((0, 0), (0, NQP - NQ), (0, 4))).reshape(G, BB * NQP, 16)

    xn55 = jnp.pad(xv, ((0, 0), (0, 0), (0, 5))).reshape(G, BB * NP, 8)
    h55 = h.reshape(G, BB * NP, NF)
    hp = jnp.pad(h, ((0, 0), (0, NJ - NP), (0, 0))).reshape(G, BB * NJ, NF)
    hB = jnp.pad(
        jnp.pad(h, ((0, 0), (0, NJ - NP), (0, 0)))
        .reshape(B, NQ, 4 * NF), ((0, 0), (0, NQP - NQ), (0, 0))
    ).reshape(G, BB * NQP, 4 * NF)
    dsp = jnp.pad(d_static, ((0, 0), (0, 0), (0, NJ - (NP - 1))))
    dsbP = dsp.reshape(G, RT, 4).astype(BF)

    eye4 = jnp.eye(4, dtype=F32)
    Wh1 = We1[:NF]
    Wh2 = We1[NF:2 * NF]
    wd = We1[2 * NF:2 * NF + 1]
    ws = We1[2 * NF + 1:2 * NF + 2]
    W12 = jnp.concatenate(
        [jnp.kron(eye4, wd), jnp.kron(eye4, ws),
         jnp.tile(be1.reshape(1, NH), (1, 4)),
         jnp.zeros((3, 4 * NH), F32)], axis=0)
    consts = dict(
        R16=jnp.asarray(_R16, BF), R55=jnp.asarray(_R55, BF),
        S2=jnp.asarray(_S2, BF), maskp=jnp.asarray(_MASKP, BF),
        wmask4=jnp.asarray(_WMASK4, BF), selmask=jnp.asarray(_SELMASK, BF),
        F16=jnp.asarray(_F16),
        Wh1t=jnp.tile(Wh1, (1, 4)).astype(BF),
        Wh2b=jnp.kron(eye4, Wh2).astype(BF),
        W12=W12.astype(BF),
        We2b=jnp.kron(eye4, We2).astype(BF),
        be2t=jnp.tile(be2.reshape(1, NF), (1, 4)),
        Wab=jnp.kron(eye4, Wa).astype(BF),
        ba=ba.reshape(1, 1),
        G4=jnp.kron(eye4, jnp.ones((1, NF), F32)).astype(BF),
        Wc1b=jnp.kron(eye4, Wc1).astype(BF),
        bc1t=jnp.tile(bc1.reshape(1, NH), (1, 4)),
        Wc2b=jnp.kron(eye4, Wc2).astype(BF),
        F4=jnp.tile(jnp.eye(NF, dtype=F32), (4, 1)).astype(BF),
        Wn1=Wn1.astype(BF), bn1=bn1.reshape(1, NH),
        Wn2=Wn2.astype(BF), bn2=bn2.reshape(1, NF),
    )

    def batch_spec(shp):
        return pl.BlockSpec((1,) + shp, lambda b: (b, 0, 0))

    def const_spec(arr):
        return pl.BlockSpec(arr.shape, lambda b: (0, 0))

    batch_args = (xn16.astype(BF), xjB16.astype(BF), xn55, h55,
                  hp.astype(BF), hB.astype(BF), dsbP)
    batch_shapes = ((BB * NJ, 16), (BB * NQP, 16), (BB * NP, 8), (BB * NP, NF),
                    (BB * NJ, NF), (BB * NQP, 4 * NF), (RT, 4))
    const_args = tuple(consts.values())

    in_specs = ([batch_spec(s) for s in batch_shapes]
                + [const_spec(a) for a in const_args])
    out_specs = (batch_spec((BB * NP, 8)), batch_spec((BB * NP, NF)))
    out_shape = (jax.ShapeDtypeStruct((G, BB * NP, 8), F32),
                 jax.ShapeDtypeStruct((G, BB * NP, NF), F32))

    xo, ho = pl.pallas_call(
        _body, grid=(G,), in_specs=in_specs, out_specs=out_specs,
        out_shape=out_shape,
        compiler_params=pltpu.CompilerParams(
            dimension_semantics=("parallel",)))(*batch_args, *const_args)
    return (xo.reshape(B, NP, 8)[:, :, :3], ho.reshape(B, NP, NF))
